# Initial kernel scaffold; baseline (speedup 1.0000x reference)
import jax
import jax.numpy as jnp
from jax.experimental import pallas as pl


def kernel(x, pos_table):
    raise NotImplementedError("write your pallas kernel here")


# speedup vs baseline: 2.3720x; 2.3720x over previous
import jax
import jax.numpy as jnp
from jax.experimental import pallas as pl

_ROWS_PER_BLOCK = 512


def _add_body(x_ref, p_ref, o_ref):
    o_ref[...] = x_ref[...] + p_ref[...]


def kernel(x, pos_table):
    n, d = x.shape
    grid = (n // _ROWS_PER_BLOCK,)
    spec = pl.BlockSpec((_ROWS_PER_BLOCK, d), lambda i: (i, 0))
    return pl.pallas_call(
        _add_body,
        out_shape=jax.ShapeDtypeStruct((n, d), x.dtype),
        grid=grid,
        in_specs=[spec, spec],
        out_specs=spec,
    )(x, pos_table)
